# baseline (speedup 1.0000x reference)
import functools

import jax
import jax.numpy as jnp
from jax import lax
from jax.experimental import pallas as pl
from jax.experimental.pallas import tpu as pltpu
from jax.experimental.pallas import tpu_sc as plsc

L = 16
NW = 32
CHUNK = 64
NBUF = 3


def _seg_sum_body(data_hbm, offsets_hbm, out_hbm, off_v, acc,
                  bufs, segs, sems,
                  *, n_rows, n_seg, d, spw, off_slice):
    nlanes = d // L
    spw_last = n_seg - (NW - 1) * spw
    search_iters = max(1, (spw - 1).bit_length())

    cid = lax.axis_index("c")
    sid = lax.axis_index("s")
    wid = cid * 16 + sid
    s0 = wid * spw
    s1 = jnp.minimum(s0 + spw, n_seg)
    sc_base = cid * 16 * spw
    acc_rows = 16 * spw + 1

    a0 = (s0 // 8) * 8
    delta = s0 - a0
    pltpu.sync_copy(offsets_hbm.at[pl.ds(a0, off_slice)], off_v)

    def off(i):
        return off_v[pl.ds(i, L)][0]

    zero = jnp.zeros((L,), jnp.float32)

    def zero_body(sl, _):
        for k in range(nlanes):
            bufs[0][sl, pl.ds(k * L, L)] = zero
        return 0

    lax.fori_loop(0, CHUNK, zero_body, 0)
    nzero = (spw + CHUNK - 1) // CHUNK + 1
    for i in range(nzero):
        zstart = jnp.minimum(sid * spw + i * CHUNK, acc_rows - CHUNK)
        pltpu.sync_copy(bufs[0], acc.at[pl.ds(zstart, CHUNK)])
    plsc.subcore_barrier()

    r_begin = off(delta)
    r_end = off(delta + (s1 - s0))
    nchunks = (r_end - r_begin + (CHUNK - 1)) // CHUNK

    lane = lax.iota(jnp.int32, L)

    def chunk_start(g):
        return jnp.minimum(r_begin + g * CHUNK, n_rows - CHUNK)

    for q in range(NBUF - 1):
        @pl.when(nchunks > q)
        def _(q=q):
            pltpu.async_copy(data_hbm.at[pl.ds(chunk_start(q), CHUNK)],
                             bufs[q], sems[q])

    l_lo = delta
    l_hi = delta + (s1 - s0)
    l_adj = a0 - sc_base

    def process(g, q):
        base = r_begin + g * CHUNK
        start = chunk_start(g)
        buf, seg_v, sem = bufs[q], segs[q], sems[q]

        @pl.when(g < nchunks)
        def _():
            pltpu.make_async_copy(data_hbm.at[pl.ds(start, CHUNK)], buf,
                                  sem).wait()

        @pl.when(g + NBUF - 1 < nchunks)
        def _():
            qn = (q + NBUF - 1) % NBUF
            pltpu.async_copy(
                data_hbm.at[pl.ds(chunk_start(g + NBUF - 1), CHUNK)],
                bufs[qn], sems[qn])

        for v in range(CHUNK // L):
            rows = start + v * L + lane
            lo_v = l_lo + jnp.zeros((L,), jnp.int32)
            hi_v = l_hi + jnp.zeros((L,), jnp.int32)
            for _it in range(search_iters):
                mid = lax.shift_right_arithmetic(lo_v + hi_v, 1)
                vals = plsc.load_gather(off_v, [mid])
                pred = vals <= rows
                lo_v = jnp.where(pred, mid, lo_v)
                hi_v = jnp.where(pred, hi_v, mid)
            valid = (rows >= base) & (rows < r_end)
            seg_v[pl.ds(v * L, L)] = jnp.where(valid, lo_v + l_adj,
                                               acc_rows - 1)

        @pl.when(g < nchunks)
        def _():
            pltpu.sync_copy(buf, acc.at[seg_v], add=True)

    def trip_body(t, _):
        for q in range(NBUF):
            process(NBUF * t + q, q)
        return 0

    lax.fori_loop(0, (nchunks + NBUF - 1) // NBUF, trip_body, 0)

    plsc.subcore_barrier()

    loc0 = sid * spw

    @pl.when(wid < NW - 1)
    def _():
        pltpu.sync_copy(acc.at[pl.ds(loc0, spw)], out_hbm.at[pl.ds(s0, spw)])

    @pl.when(wid == NW - 1)
    def _():
        pltpu.sync_copy(acc.at[pl.ds(loc0, spw_last)],
                        out_hbm.at[pl.ds(s0, spw_last)])


def _segment_sum_sc(data, offsets):
    n_rows, d = data.shape
    n_seg = offsets.shape[0] - 1
    spw = (n_seg + NW - 1) // NW
    off_slice = ((7 + spw + 1 + 15) + 7) // 8 * 8
    a0_max = ((NW - 1) * spw) // 8 * 8
    pad = max(0, a0_max + off_slice - (n_seg + 1))
    offsets_padded = jnp.pad(offsets, (0, pad))

    mesh = plsc.VectorSubcoreMesh(core_axis_name="c", subcore_axis_name="s")
    kern = pl.kernel(
        functools.partial(_seg_sum_body, n_rows=n_rows, n_seg=n_seg, d=d,
                          spw=spw, off_slice=off_slice),
        mesh=mesh,
        compiler_params=pltpu.CompilerParams(use_tc_tiling_on_sc=False,
                                             needs_layout_passes=False),
        out_type=jax.ShapeDtypeStruct((n_seg, d), jnp.float32),
        scratch_types=[
            pltpu.VMEM((off_slice,), jnp.int32),
            pltpu.VMEM_SHARED((16 * spw + 1, d), jnp.float32),
            [pltpu.VMEM((CHUNK, d), jnp.float32) for _ in range(NBUF)],
            [pltpu.VMEM((CHUNK,), jnp.int32) for _ in range(NBUF)],
            [pltpu.SemaphoreType.DMA for _ in range(NBUF)],
        ],
    )
    return kern(data, offsets_padded)


def kernel(data, reduce, lengths, indices, offsets, axis, unsafe, initial, out):
    res = _segment_sum_sc(data, offsets.astype(jnp.int32))
    return res + jnp.asarray(initial, dtype=data.dtype)


# speedup vs baseline: 1.0269x; 1.0269x over previous
import functools

import jax
import jax.numpy as jnp
from jax import lax
from jax.experimental import pallas as pl
from jax.experimental.pallas import tpu as pltpu
from jax.experimental.pallas import tpu_sc as plsc

L = 16
NW = 32
CHUNK = 64
NBUF = 3


def _seg_sum_body(data_hbm, offsets_hbm, out_hbm, off_v, acc,
                  bufs, segs, sems,
                  *, n_rows, n_seg, d, spw, off_slice):
    nlanes = d // L
    spw_last = n_seg - (NW - 1) * spw
    search_iters = max(1, (spw - 1).bit_length())

    cid = lax.axis_index("c")
    sid = lax.axis_index("s")
    wid = cid * 16 + sid
    s0 = wid * spw
    s1 = jnp.minimum(s0 + spw, n_seg)
    sc_base = cid * 16 * spw
    acc_rows = 16 * spw + 1

    a0 = (s0 // 8) * 8
    delta = s0 - a0
    pltpu.sync_copy(offsets_hbm.at[pl.ds(a0, off_slice)], off_v)

    def off(i):
        return off_v[pl.ds(i, L)][0]

    zero = jnp.zeros((L,), jnp.float32)

    def zero_body(sl, _):
        for k in range(nlanes):
            bufs[0][sl, pl.ds(k * L, L)] = zero
        return 0

    lax.fori_loop(0, CHUNK, zero_body, 0)
    zbase = sid * spw
    for i in range(spw // CHUNK):
        pltpu.sync_copy(bufs[0], acc.at[pl.ds(zbase + i * CHUNK, CHUNK)])
    rem = spw % CHUNK
    if rem:
        pltpu.sync_copy(bufs[0].at[pl.ds(0, rem)],
                        acc.at[pl.ds(zbase + (spw // CHUNK) * CHUNK, rem)])

    r_begin = off(delta)
    r_end = off(delta + (s1 - s0))
    nchunks = (r_end - r_begin + (CHUNK - 1)) // CHUNK

    lane = lax.iota(jnp.int32, L)

    def chunk_start(g):
        return jnp.minimum(r_begin + g * CHUNK, n_rows - CHUNK)

    for q in range(NBUF - 1):
        @pl.when(nchunks > q)
        def _(q=q):
            pltpu.async_copy(data_hbm.at[pl.ds(chunk_start(q), CHUNK)],
                             bufs[q], sems[q])

    l_lo = delta
    l_hi = delta + (s1 - s0)
    l_adj = a0 - sc_base

    def process(g, q):
        base = r_begin + g * CHUNK
        start = chunk_start(g)
        buf, seg_v, sem = bufs[q], segs[q], sems[q]

        @pl.when(g < nchunks)
        def _():
            pltpu.make_async_copy(data_hbm.at[pl.ds(start, CHUNK)], buf,
                                  sem).wait()

        @pl.when(g + NBUF - 1 < nchunks)
        def _():
            qn = (q + NBUF - 1) % NBUF
            pltpu.async_copy(
                data_hbm.at[pl.ds(chunk_start(g + NBUF - 1), CHUNK)],
                bufs[qn], sems[qn])

        for v in range(CHUNK // L):
            rows = start + v * L + lane
            lo_v = l_lo + jnp.zeros((L,), jnp.int32)
            hi_v = l_hi + jnp.zeros((L,), jnp.int32)
            for _it in range(search_iters):
                mid = lax.shift_right_arithmetic(lo_v + hi_v, 1)
                vals = plsc.load_gather(off_v, [mid])
                pred = vals <= rows
                lo_v = jnp.where(pred, mid, lo_v)
                hi_v = jnp.where(pred, hi_v, mid)
            valid = (rows >= base) & (rows < r_end)
            seg_v[pl.ds(v * L, L)] = jnp.where(valid, lo_v + l_adj,
                                               acc_rows - 1)

        @pl.when(g < nchunks)
        def _():
            pltpu.sync_copy(buf, acc.at[seg_v], add=True)

    def trip_body(t, _):
        for q in range(NBUF):
            process(NBUF * t + q, q)
        return 0

    lax.fori_loop(0, (nchunks + NBUF - 1) // NBUF, trip_body, 0)

    loc0 = sid * spw

    @pl.when(wid < NW - 1)
    def _():
        pltpu.sync_copy(acc.at[pl.ds(loc0, spw)], out_hbm.at[pl.ds(s0, spw)])

    @pl.when(wid == NW - 1)
    def _():
        pltpu.sync_copy(acc.at[pl.ds(loc0, spw_last)],
                        out_hbm.at[pl.ds(s0, spw_last)])


def _segment_sum_sc(data, offsets):
    n_rows, d = data.shape
    n_seg = offsets.shape[0] - 1
    spw = (n_seg + NW - 1) // NW
    off_slice = ((7 + spw + 1 + 15) + 7) // 8 * 8
    a0_max = ((NW - 1) * spw) // 8 * 8
    pad = max(0, a0_max + off_slice - (n_seg + 1))
    offsets_padded = jnp.pad(offsets, (0, pad))

    mesh = plsc.VectorSubcoreMesh(core_axis_name="c", subcore_axis_name="s")
    kern = pl.kernel(
        functools.partial(_seg_sum_body, n_rows=n_rows, n_seg=n_seg, d=d,
                          spw=spw, off_slice=off_slice),
        mesh=mesh,
        compiler_params=pltpu.CompilerParams(use_tc_tiling_on_sc=False,
                                             needs_layout_passes=False),
        out_type=jax.ShapeDtypeStruct((n_seg, d), jnp.float32),
        scratch_types=[
            pltpu.VMEM((off_slice,), jnp.int32),
            pltpu.VMEM_SHARED((16 * spw + 1, d), jnp.float32),
            [pltpu.VMEM((CHUNK, d), jnp.float32) for _ in range(NBUF)],
            [pltpu.VMEM((CHUNK,), jnp.int32) for _ in range(NBUF)],
            [pltpu.SemaphoreType.DMA for _ in range(NBUF)],
        ],
    )
    return kern(data, offsets_padded)


def kernel(data, reduce, lengths, indices, offsets, axis, unsafe, initial, out):
    res = _segment_sum_sc(data, offsets.astype(jnp.int32))
    return res + jnp.asarray(initial, dtype=data.dtype)
